# baseline (speedup 1.0000x reference)
import functools

import jax
import jax.numpy as jnp
from jax import lax
from jax.experimental import pallas as pl
from jax.experimental.pallas import tpu as pltpu
from jax.experimental.pallas import tpu_sc as plsc

_N = 10000
_E = 160000


def _tc_layer_body(*refs, relu_in, dh, nx):
    x_refs = refs[:nx]
    norm_ref, w_ref, out0_ref, out1_ref = refs[nx:]
    if nx == 1:
        x = x_refs[0][...]
    else:
        x = jnp.concatenate([r[...] for r in x_refs], axis=1)
    nrm = norm_ref[...]
    if relu_in:
        x = jnp.maximum(x * nrm, 0.0)
    y = jnp.dot(x, w_ref[...], preferred_element_type=jnp.float32)
    y = y * nrm
    out0_ref[...] = y[:, :dh]
    out1_ref[...] = y[:, dh:]


def _tc_layer(xs, norm, w, relu_in):
    n = xs[0].shape[0]
    dout = w.shape[1]
    dh = dout // 2
    blk = 2000
    return pl.pallas_call(
        functools.partial(_tc_layer_body, relu_in=relu_in, dh=dh, nx=len(xs)),
        grid=(n // blk,),
        in_specs=[
            pl.BlockSpec((blk, x.shape[1]), lambda i: (i, 0)) for x in xs
        ] + [
            pl.BlockSpec((blk, 1), lambda i: (i, 0)),
            pl.BlockSpec(w.shape, lambda i: (0, 0)),
        ],
        out_specs=[
            pl.BlockSpec((blk, dh), lambda i: (i, 0)),
            pl.BlockSpec((blk, dh), lambda i: (i, 0)),
        ],
        out_shape=[
            jax.ShapeDtypeStruct((n, dh), jnp.float32),
            jax.ShapeDtypeStruct((n, dh), jnp.float32),
        ],
    )(*xs, norm, w)


def _scale_body(x0_ref, x1_ref, norm_ref, o_ref):
    x = jnp.concatenate([x0_ref[...], x1_ref[...]], axis=1)
    o_ref[...] = x * norm_ref[...]


def _final_scale(x0, x1, norm):
    n, dh = x0.shape
    blk = 2000
    return pl.pallas_call(
        _scale_body,
        grid=(n // blk,),
        in_specs=[
            pl.BlockSpec((blk, dh), lambda i: (i, 0)),
            pl.BlockSpec((blk, dh), lambda i: (i, 0)),
            pl.BlockSpec((blk, 1), lambda i: (i, 0)),
        ],
        out_specs=pl.BlockSpec((blk, 2 * dh), lambda i: (i, 0)),
        out_shape=jax.ShapeDtypeStruct((n, 2 * dh), jnp.float32),
    )(x0, x1, norm)


_K = 80
_NSTG = 5
_CPS = 25


def _make_sc_agg(d2):
    rpt = 624

    mesh = plsc.VectorSubcoreMesh(core_axis_name="c", subcore_axis_name="s")

    @functools.partial(
        pl.kernel,
        mesh=mesh,
        compiler_params=pltpu.CompilerParams(use_tc_tiling_on_sc=(d2 % 128 == 0)),
        out_type=[
            jax.ShapeDtypeStruct((_N, d2), jnp.float32),
            jax.ShapeDtypeStruct((_N, d2), jnp.float32),
        ],
        scratch_types=[
            pltpu.VMEM((_CPS, _K), jnp.int32),
            pltpu.VMEM((_CPS, _K), jnp.int32),
            pltpu.VMEM((3, _K, d2), jnp.float32),
            pltpu.VMEM_SHARED((_N, d2), jnp.float32),
            pltpu.SemaphoreType.DMA,
            pltpu.SemaphoreType.DMA,
            pltpu.SemaphoreType.DMA,
            pltpu.SemaphoreType.DMA,
            pltpu.SemaphoreType.DMA,
            pltpu.SemaphoreType.DMA,
        ],
    )
    def agg(g0_hbm, g1_hbm, src_hbm, dst_hbm, zero_hbm, out0_hbm, out1_hbm,
            src_v, dst_v, rows_v, acc_sh,
            gsem0, gsem1, gsem2, ssem0, ssem1, ssem2):
        c = lax.axis_index("c")
        s = lax.axis_index("s")
        row0 = s * rpt
        tail = 16 * rpt

        pltpu.sync_copy(zero_hbm, acc_sh.at[pl.ds(row0, rpt)])

        @pl.when(s == 15)
        def _():
            pltpu.sync_copy(zero_hbm.at[pl.ds(0, 16)],
                            acc_sh.at[pl.ds(tail, 16)])

        plsc.subcore_barrier()

        def run(g_hbm, out_hbm):
            gsems = (gsem0, gsem1, gsem2)
            ssems = (ssem0, ssem1, ssem2)

            def gstart(j, b):
                pltpu.async_copy(g_hbm.at[src_v.at[j]], rows_v.at[b], gsems[b])

            def gwait(j, b):
                pltpu.make_async_copy(
                    g_hbm.at[src_v.at[j]], rows_v.at[b], gsems[b]).wait()

            def sstart(j, b):
                pltpu.async_copy(rows_v.at[b], acc_sh.at[dst_v.at[j]],
                                 ssems[b], add=True)

            def swait(j, b):
                pltpu.make_async_copy(
                    rows_v.at[b], acc_sh.at[dst_v.at[j]], ssems[b]).wait()

            for blk in range(_NSTG):
                pltpu.sync_copy(src_hbm.at[s, blk], src_v)
                pltpu.sync_copy(dst_hbm.at[s, blk], dst_v)
                gstart(0, 0)
                gstart(1, 1)
                gwait(0, 0)
                sstart(0, 0)
                gstart(2, 2)
                gwait(1, 1)
                sstart(1, 1)
                swait(0, 0)
                gstart(3, 0)

                def body(i, carry):
                    for dj in range(3):
                        j = 3 * i + 2 + dj
                        b = (2 + dj) % 3
                        gwait(j, b)
                        sstart(j, b)
                        swait(j - 1, (1 + dj) % 3)
                        gstart(j + 2, (4 + dj) % 3)
                    return carry

                lax.fori_loop(0, (_CPS - 4) // 3, body, 0)
                for j in range(_CPS - 2, _CPS):
                    b = j % 3
                    gwait(j, b)
                    sstart(j, b)
                    swait(j - 1, (j - 1) % 3)
                swait(_CPS - 1, (_CPS - 1) % 3)

            plsc.subcore_barrier()
            pltpu.sync_copy(acc_sh.at[pl.ds(row0, rpt)],
                            out_hbm.at[pl.ds(row0, rpt)])

            @pl.when(s == 15)
            def _():
                pltpu.sync_copy(acc_sh.at[pl.ds(tail, 16)],
                                out_hbm.at[pl.ds(tail, 16)])

        @pl.when(c == 0)
        def _():
            run(g0_hbm, out0_hbm)

        @pl.when(c == 1)
        def _():
            run(g1_hbm, out1_hbm)

    return agg


_sc_agg_128 = _make_sc_agg(128)
_sc_agg_32 = _make_sc_agg(32)


def kernel(features, norm, edge_index, W0, W1, W2):
    src = edge_index[0].reshape(16, _NSTG, _CPS, _K)
    dst = edge_index[1].reshape(16, _NSTG, _CPS, _K)
    z128 = jnp.zeros((624, 128), jnp.float32)
    z32 = jnp.zeros((624, 32), jnp.float32)

    g0, g1 = _tc_layer([features], norm, W0, relu_in=False)
    h0, h1 = _sc_agg_128(g0, g1, src, dst, z128)
    g0, g1 = _tc_layer([h0, h1], norm, W1, relu_in=True)
    h0, h1 = _sc_agg_128(g0, g1, src, dst, z128)
    g0, g1 = _tc_layer([h0, h1], norm, W2, relu_in=True)
    h0, h1 = _sc_agg_32(g0, g1, src, dst, z32)
    return _final_scale(h0, h1, norm)
